# speedup vs baseline: 1.2750x; 1.2750x over previous
import functools
import math

import jax
import jax.numpy as jnp
from jax.experimental import pallas as pl

NS = 1
NR = 8
KR = 2
D_MODEL = 1024
D_FF = 1024
N_TOKENS = 2048
NE = NS + NR


def _routing_body(x_ref, c_ref, b_ref, g_ref):
    x = x_ref[...]
    c = c_ref[...]
    scores = jax.nn.sigmoid(
        jnp.dot(x, c.T, preferred_element_type=jnp.float32)) + b_ref[...][None, :]
    lane = jax.lax.broadcasted_iota(jnp.int32, scores.shape, 1)
    valid = lane < NR
    neg = jnp.float32(-1e30)
    s = jnp.where(valid, scores, neg)
    m1 = jnp.max(s, axis=1, keepdims=True)
    idx1 = jnp.min(jnp.where(s == m1, lane, 99), axis=1, keepdims=True)
    sel1 = lane == idx1
    s2 = jnp.where(sel1, neg, s)
    m2 = jnp.max(s2, axis=1, keepdims=True)
    idx2 = jnp.min(jnp.where(s2 == m2, lane, 99), axis=1, keepdims=True)
    sel2 = lane == idx2
    denom = jnp.clip(m1 + m2, 1e-8, None)
    g = jnp.where(sel1, m1, 0.0) + jnp.where(sel2, m2, 0.0)
    g_ref[...] = jnp.where(valid, g / denom, 0.0)


def _ffn_body(x_ref, g_ref, w1_ref, b1_ref, w2_ref, b2_ref, o_ref):
    j = pl.program_id(0)
    x = x_ref[...]
    w1 = w1_ref[0]
    w2 = w2_ref[0]
    h = jnp.maximum(
        jnp.dot(x, w1, preferred_element_type=jnp.float32) + b1_ref[0, 0][None, :],
        0.0)
    y = jnp.dot(h, w2, preferred_element_type=jnp.float32) + b2_ref[0, 0][None, :]
    lane = jax.lax.broadcasted_iota(jnp.int32, g_ref.shape, 1)
    gcol = jnp.sum(jnp.where(lane == j - 1, g_ref[...], 0.0), axis=1,
                   keepdims=True)
    gcol = jnp.where(j == 0, 1.0, gcol)
    base = jnp.where(j == 0, x, o_ref[...])
    o_ref[...] = base + y * gcol


def kernel(u, centroids, bias, shared_W1, shared_b1, shared_W2, shared_b2,
           routed_W1, routed_b1, routed_W2, routed_b2):
    Bq, Sq, D = u.shape
    flat = u.reshape(-1, D)
    n = flat.shape[0]

    c16 = jnp.zeros((16, D), jnp.float32).at[:NR].set(centroids)
    b16 = jnp.zeros((16,), jnp.float32).at[:NR].set(bias)

    gates = pl.pallas_call(
        _routing_body,
        out_shape=jax.ShapeDtypeStruct((n, 16), jnp.float32),
    )(flat, c16, b16)

    w1 = jnp.concatenate([shared_W1, routed_W1], axis=0)
    w2 = jnp.concatenate([shared_W2, routed_W2], axis=0)
    b1 = jnp.concatenate([shared_b1, routed_b1], axis=0).reshape(NE, 1, D_FF)
    b2 = jnp.concatenate([shared_b2, routed_b2], axis=0).reshape(NE, 1, D_MODEL)

    out = pl.pallas_call(
        _ffn_body,
        grid=(NE,),
        in_specs=[
            pl.BlockSpec((n, D), lambda j: (0, 0)),
            pl.BlockSpec((n, 16), lambda j: (0, 0)),
            pl.BlockSpec((1, D_MODEL, D_FF), lambda j: (j, 0, 0)),
            pl.BlockSpec((1, 1, D_FF), lambda j: (j, 0, 0)),
            pl.BlockSpec((1, D_FF, D_MODEL), lambda j: (j, 0, 0)),
            pl.BlockSpec((1, 1, D_MODEL), lambda j: (j, 0, 0)),
        ],
        out_specs=pl.BlockSpec((n, D), lambda j: (0, 0)),
        out_shape=jax.ShapeDtypeStruct((n, D), jnp.float32),
    )(flat, gates, w1, b1, w2, b2)

    return out.reshape(Bq, Sq, D)
